# baseline (speedup 1.0000x reference)
import functools

import jax
import jax.numpy as jnp
from jax import lax
from jax.experimental import pallas as pl
from jax.experimental.pallas import tpu as pltpu
from jax.experimental.pallas import tpu_sc as plsc

B = 4096
EMB = 32
HID = 64
NW = 32
BPW = B // NW
RB = 256
GRID = B // RB


def _sc_gather_body(uid, mid, gid, ut, mt, gt, bt32,
                    uo, mo, go, bo,
                    idx_v, hi_v, rows_v, bias_v, sem):
    wid = lax.axis_index("s") * 2 + lax.axis_index("c")
    base = wid * BPW
    pltpu.sync_copy(uid.at[pl.ds(base, BPW)], idx_v)
    pltpu.async_copy(ut.at[idx_v], rows_v, sem).wait()
    pltpu.sync_copy(rows_v, uo.at[pl.ds(base, BPW)])
    pltpu.sync_copy(mid.at[pl.ds(base, BPW)], idx_v)
    pltpu.async_copy(mt.at[idx_v], rows_v, sem).wait()
    pltpu.sync_copy(rows_v, mo.at[pl.ds(base, BPW)])
    for k in range(BPW // 16):
        idx16 = idx_v[pl.ds(k * 16, 16)]
        hi_v[pl.ds(k * 16, 16)] = lax.shift_right_logical(idx16, 5)
    pltpu.async_copy(bt32.at[hi_v], rows_v, sem).wait()
    for k in range(BPW // 16):
        idx16 = idx_v[pl.ds(k * 16, 16)]
        off = lax.bitwise_and(idx16, 31)
        row = lax.iota(jnp.int32, 16) + k * 16
        bias_v[pl.ds(k * 16, 16)] = plsc.load_gather(rows_v, [row, off])
    pltpu.sync_copy(bias_v, bo.at[pl.ds(base, BPW)])
    pltpu.sync_copy(gid.at[pl.ds(base, BPW)], idx_v)
    pltpu.async_copy(gt.at[idx_v], rows_v, sem).wait()
    pltpu.sync_copy(rows_v, go.at[pl.ds(base, BPW)])


@functools.cache
def _sc_gather():
    return pl.kernel(
        _sc_gather_body,
        out_type=(
            jax.ShapeDtypeStruct((B, EMB), jnp.float32),
            jax.ShapeDtypeStruct((B, EMB), jnp.float32),
            jax.ShapeDtypeStruct((B, EMB), jnp.float32),
            jax.ShapeDtypeStruct((B,), jnp.float32),
        ),
        mesh=plsc.VectorSubcoreMesh(core_axis_name="c", subcore_axis_name="s"),
        compiler_params=pltpu.CompilerParams(use_tc_tiling_on_sc=False,
                                             needs_layout_passes=False),
        scratch_types=(
            pltpu.VMEM((BPW,), jnp.int32),
            pltpu.VMEM((BPW,), jnp.int32),
            pltpu.VMEM((BPW, EMB), jnp.float32),
            pltpu.VMEM((BPW,), jnp.float32),
            pltpu.SemaphoreType.DMA,
        ),
    )


def _tc_dense_body(ue, me, ge, mb, w1, b1, w2r, b2, a_out, b_out):
    u = ue[...]
    m = me[...]
    g = ge[...]
    h = (jnp.dot(u, w1[0:EMB, :], preferred_element_type=jnp.float32)
         + jnp.dot(m, w1[EMB:2 * EMB, :], preferred_element_type=jnp.float32)
         + jnp.dot(g, w1[2 * EMB:3 * EMB, :], preferred_element_type=jnp.float32)
         + b1[...])
    h = jnp.maximum(h, 0.0)
    a_out[...] = jnp.sum(h * w2r[...], axis=1, keepdims=True) + b2[...]
    wide = (jnp.sum(u, axis=1, keepdims=True)
            + jnp.sum(m, axis=1, keepdims=True)
            + jnp.sum(g, axis=1, keepdims=True))
    b_out[...] = wide + mb[...]


def _tc_dense(ue, me, ge, mb, w1, b1r, w2r, b2):
    return pl.pallas_call(
        _tc_dense_body,
        out_shape=(
            jax.ShapeDtypeStruct((B, 1), jnp.float32),
            jax.ShapeDtypeStruct((B, 1), jnp.float32),
        ),
    )(ue, me, ge, mb, w1, b1r, w2r, b2)


def _tc_writer_body(a_ref, brow_ref, out_ref):
    out_ref[...] = a_ref[...] + brow_ref[...]


def _tc_writer(a, brow):
    return pl.pallas_call(
        _tc_writer_body,
        grid=(GRID,),
        in_specs=[
            pl.BlockSpec((RB, 1), lambda i: (i, 0)),
            pl.BlockSpec((1, B), lambda i: (0, 0)),
        ],
        out_specs=pl.BlockSpec((RB, B), lambda i: (i, 0)),
        out_shape=jax.ShapeDtypeStruct((B, B), jnp.float32),
    )(a, brow)


def kernel(user_ids, movie_ids, genre_ids, user_table, movie_table,
           genre_table, movie_bias_table, W1, b1, W2, b2):
    uid = user_ids.astype(jnp.int32)
    mid = movie_ids.astype(jnp.int32)
    gid = genre_ids.astype(jnp.int32)
    bt32 = movie_bias_table.reshape(-1, EMB)
    ue, me, ge, mb = _sc_gather()(uid, mid, gid, user_table, movie_table,
                                  genre_table, bt32)
    mb = mb.reshape(B, 1)
    b1r = b1.reshape(1, HID)
    w2r = W2.reshape(1, HID)
    b2r = b2.reshape(1, 1)
    a, bcol = _tc_dense(ue, me, ge, mb, W1, b1r, w2r, b2r)
    brow = bcol.reshape(1, B)
    return _tc_writer(a, brow)


# speedup vs baseline: 4.9124x; 4.9124x over previous
import functools

import jax
import jax.numpy as jnp
from jax import lax
from jax.experimental import pallas as pl
from jax.experimental.pallas import tpu as pltpu
from jax.experimental.pallas import tpu_sc as plsc

B = 4096
EMB = 32
HID = 64
NC = 2
NS = 16
NW = NC * NS
BPW = B // NW
WAVE = 16
NGENRE = 1000
RB = 256
GRID = B // RB


def _iota16():
    return lax.iota(jnp.int32, 16)


def _splat16(x):
    return jnp.full((16,), x, jnp.int32)


def _window_gather(idx_v, table_t, out_t, win_v, ebuf, semw, base):
    iota = _iota16()

    def wave(w, _):
        idx16 = idx_v[pl.ds(w * WAVE, 16)]
        copies = []
        for t in range(WAVE):
            r = idx16[t]
            j = pl.multiple_of((r >> 7) * 128, 128)
            copies.append(pltpu.async_copy(table_t.at[:, pl.ds(j, 128)],
                                           win_v.at[t], semw))
        for t in range(WAVE):
            copies[t].wait()
            r = idx16[t]
            cc = r & 127
            col = _splat16(w * WAVE + t)
            lo = plsc.load_gather(win_v, [_splat16(t), iota, _splat16(cc)])
            hi = plsc.load_gather(win_v, [_splat16(t), iota + 16,
                                          _splat16(cc)])
            plsc.store_scatter(ebuf, [iota, col], lo)
            plsc.store_scatter(ebuf, [iota + 16, col], hi)
        return 0

    lax.fori_loop(0, BPW // WAVE, wave, 0, unroll=False)
    obase = pl.multiple_of(base, 128)
    pltpu.sync_copy(ebuf, out_t.at[:, pl.ds(obase, BPW)])


def _sc_gather_body(uid, mid, gid, ut_t, mt_t, gt_t, bias1d,
                    uo_t, mo_t, go_t, bo,
                    idx_v, win_v, ebuf, gvm, bwin, bbuf, semw):
    c = lax.axis_index("c")
    s = lax.axis_index("s")
    wid = s * NC + c
    base = wid * BPW
    iota = _iota16()

    pltpu.sync_copy(uid.at[pl.ds(base, BPW)], idx_v)
    _window_gather(idx_v, ut_t, uo_t, win_v, ebuf, semw, base)

    pltpu.sync_copy(mid.at[pl.ds(base, BPW)], idx_v)
    _window_gather(idx_v, mt_t, mo_t, win_v, ebuf, semw, base)

    def b_wave(w, _):
        idx16 = idx_v[pl.ds(w * WAVE, 16)]
        copies = []
        for t in range(WAVE):
            r = idx16[t]
            jb = pl.multiple_of((r >> 7) * 128, 128)
            copies.append(pltpu.async_copy(bias1d.at[pl.ds(jb, 128)],
                                           bwin.at[t], semw))
        for cp in copies:
            cp.wait()
        bbuf[pl.ds(w * WAVE, 16)] = plsc.load_gather(bwin, [iota, idx16 & 127])
        return 0

    lax.fori_loop(0, BPW // WAVE, b_wave, 0, unroll=False)
    obase = pl.multiple_of(base, 128)
    pltpu.sync_copy(bbuf, bo.at[pl.ds(obase, BPW)])

    pltpu.sync_copy(gt_t, gvm)
    pltpu.sync_copy(gid.at[pl.ds(base, BPW)], idx_v)

    def g_wave(w, _):
        idx16 = idx_v[pl.ds(w * WAVE, 16)]
        col = _splat16(w * WAVE) + iota
        for e in range(EMB):
            vals = plsc.load_gather(gvm, [_splat16(e), idx16])
            plsc.store_scatter(ebuf, [_splat16(e), col], vals)
        return 0

    lax.fori_loop(0, BPW // WAVE, g_wave, 0, unroll=False)
    pltpu.sync_copy(ebuf, go_t.at[:, pl.ds(obase, BPW)])


@functools.cache
def _sc_gather():
    return pl.kernel(
        _sc_gather_body,
        out_type=(
            jax.ShapeDtypeStruct((EMB, B), jnp.float32),
            jax.ShapeDtypeStruct((EMB, B), jnp.float32),
            jax.ShapeDtypeStruct((EMB, B), jnp.float32),
            jax.ShapeDtypeStruct((B,), jnp.float32),
        ),
        mesh=plsc.VectorSubcoreMesh(core_axis_name="c", subcore_axis_name="s"),
        compiler_params=pltpu.CompilerParams(use_tc_tiling_on_sc=True,
                                             needs_layout_passes=False),
        scratch_types=(
            pltpu.VMEM((BPW,), jnp.int32),
            pltpu.VMEM((WAVE, EMB, 128), jnp.float32),
            pltpu.VMEM((EMB, BPW), jnp.float32),
            pltpu.VMEM((EMB, NGENRE), jnp.float32),
            pltpu.VMEM((WAVE, 128), jnp.float32),
            pltpu.VMEM((BPW,), jnp.float32),
            pltpu.SemaphoreType.DMA,
        ),
    )


def _tc_dense_body(ue_t, me_t, ge_t, brow, w1, b1, w2r, b2, a_out, b_out):
    u = ue_t[...]
    m = me_t[...]
    g = ge_t[...]
    dn = (((0,), (0,)), ((), ()))
    h = (lax.dot_general(u, w1[0:EMB, :], dn,
                         preferred_element_type=jnp.float32)
         + lax.dot_general(m, w1[EMB:2 * EMB, :], dn,
                           preferred_element_type=jnp.float32)
         + lax.dot_general(g, w1[2 * EMB:3 * EMB, :], dn,
                           preferred_element_type=jnp.float32)
         + b1[...])
    h = jnp.maximum(h, 0.0)
    a_out[...] = jnp.sum(h * w2r[...], axis=1, keepdims=True) + b2[...]
    wide = jnp.sum(u + m + g, axis=0, keepdims=True)
    b_out[...] = wide + brow[...]


def _tc_dense(ue_t, me_t, ge_t, brow, w1, b1r, w2r, b2r):
    return pl.pallas_call(
        _tc_dense_body,
        out_shape=(
            jax.ShapeDtypeStruct((B, 1), jnp.float32),
            jax.ShapeDtypeStruct((1, B), jnp.float32),
        ),
    )(ue_t, me_t, ge_t, brow, w1, b1r, w2r, b2r)


def _tc_writer_body(a_ref, brow_ref, out_ref):
    out_ref[...] = a_ref[...] + brow_ref[...]


def _tc_writer(a, brow):
    return pl.pallas_call(
        _tc_writer_body,
        grid=(GRID,),
        in_specs=[
            pl.BlockSpec((RB, 1), lambda i: (i, 0)),
            pl.BlockSpec((1, B), lambda i: (0, 0)),
        ],
        out_specs=pl.BlockSpec((RB, B), lambda i: (i, 0)),
        out_shape=jax.ShapeDtypeStruct((B, B), jnp.float32),
    )(a, brow)


def kernel(user_ids, movie_ids, genre_ids, user_table, movie_table,
           genre_table, movie_bias_table, W1, b1, W2, b2):
    uid = user_ids.astype(jnp.int32)
    mid = movie_ids.astype(jnp.int32)
    gid = genre_ids.astype(jnp.int32)
    ut_t = user_table.T
    mt_t = movie_table.T
    gt_t = genre_table.T
    bias1d = movie_bias_table.reshape(-1)
    ue_t, me_t, ge_t, bvals = _sc_gather()(uid, mid, gid, ut_t, mt_t, gt_t,
                                           bias1d)
    brow = bvals.reshape(1, B)
    b1r = b1.reshape(1, HID)
    w2r = W2.reshape(1, HID)
    b2r = b2.reshape(1, 1)
    a, bvec = _tc_dense(ue_t, me_t, ge_t, brow, W1, b1r, w2r, b2r)
    return _tc_writer(a, bvec)


# speedup vs baseline: 5.6971x; 1.1597x over previous
import functools

import jax
import jax.numpy as jnp
from jax import lax
from jax.experimental import pallas as pl
from jax.experimental.pallas import tpu as pltpu
from jax.experimental.pallas import tpu_sc as plsc

B = 4096
EMB = 32
HID = 64
NC = 2
NS = 16
NW = NC * NS
BPW = B // NW
WAVE = 8
NWAVES = BPW // WAVE
NGENRE = 1000
RB = 512
GRID = B // RB


def _iota16():
    return lax.iota(jnp.int32, 16)


def _splat16(x):
    return jnp.full((16,), x, jnp.int32)


def _wg_ids(idx_v, w):
    return idx_v[pl.ds(w * WAVE, 16)]


def _window_gather(idx_v, table_t, out_t, win_v, ebuf, semw, base,
                   bias1d=None, bwin=None, bbuf=None):
    iota = _iota16()

    def fire(w):
        bank = w & 1
        idx16 = _wg_ids(idx_v, w)
        for t in range(WAVE):
            r = idx16[t]
            j = pl.multiple_of((r >> 7) * 128, 128)
            pltpu.async_copy(table_t.at[:, pl.ds(j, 128)],
                             win_v.at[bank, t], semw)
            if bias1d is not None:
                pltpu.async_copy(bias1d.at[pl.ds(j, 128)],
                                 bwin.at[bank, t], semw)

    def drain_extract(w):
        bank = w & 1
        idx16 = _wg_ids(idx_v, w)
        for t in range(WAVE):
            r = idx16[t]
            j = pl.multiple_of((r >> 7) * 128, 128)
            pltpu.make_async_copy(table_t.at[:, pl.ds(j, 128)],
                                  win_v.at[bank, t], semw).wait()
            if bias1d is not None:
                pltpu.make_async_copy(bias1d.at[pl.ds(j, 128)],
                                      bwin.at[bank, t], semw).wait()
            cc = r & 127
            col = _splat16(w * WAVE + t)
            lo = plsc.load_gather(win_v, [_splat16(bank), _splat16(t), iota,
                                          _splat16(cc)])
            hi = plsc.load_gather(win_v, [_splat16(bank), _splat16(t),
                                          iota + 16, _splat16(cc)])
            plsc.store_scatter(ebuf, [iota, col], lo)
            plsc.store_scatter(ebuf, [iota + 16, col], hi)

    def bias_extract(w):
        bank = w & 1
        sel = jnp.where(iota < WAVE, iota, 0)
        idx8 = plsc.load_gather(idx_v, [_splat16(w * WAVE) + sel])
        bv = plsc.load_gather(bwin, [_splat16(bank), sel, idx8 & 127])
        plsc.store_scatter(bbuf, [_splat16(w * WAVE) + sel], bv,
                           mask=iota < WAVE)

    def body(w, _):
        fire(w + 1)
        drain_extract(w)
        if bias1d is not None:
            bias_extract(w)
        return 0

    fire(0)
    lax.fori_loop(0, NWAVES - 1, body, 0, unroll=False)
    drain_extract(NWAVES - 1)
    if bias1d is not None:
        bias_extract(NWAVES - 1)
    obase = pl.multiple_of(base, 128)
    pltpu.sync_copy(ebuf, out_t.at[:, pl.ds(obase, BPW)])


def _sc_gather_body(uid, mid, gid, ut_t, mt_t, gt_t, bias1d,
                    uo_t, mo_t, go_t, bo,
                    idx_v, win_v, ebuf, gvm, bwin, bbuf, semw, semg):
    c = lax.axis_index("c")
    s = lax.axis_index("s")
    wid = s * NC + c
    base = wid * BPW
    iota = _iota16()
    obase = pl.multiple_of(base, 128)

    gstage = pltpu.async_copy(gt_t, gvm, semg)

    pltpu.sync_copy(uid.at[pl.ds(base, BPW)], idx_v.at[pl.ds(0, BPW)])
    _window_gather(idx_v, ut_t, uo_t, win_v, ebuf, semw, base)

    pltpu.sync_copy(mid.at[pl.ds(base, BPW)], idx_v.at[pl.ds(0, BPW)])
    _window_gather(idx_v, mt_t, mo_t, win_v, ebuf, semw, base,
                   bias1d=bias1d, bwin=bwin, bbuf=bbuf)
    pltpu.sync_copy(bbuf, bo.at[pl.ds(obase, BPW)])

    gstage.wait()
    pltpu.sync_copy(gid.at[pl.ds(base, BPW)], idx_v.at[pl.ds(0, BPW)])

    def g_wave(w, _):
        idx16 = idx_v[pl.ds(w * 16, 16)]
        col = _splat16(w * 16) + iota
        for e in range(EMB):
            vals = plsc.load_gather(gvm, [_splat16(e), idx16])
            plsc.store_scatter(ebuf, [_splat16(e), col], vals)
        return 0

    lax.fori_loop(0, BPW // 16, g_wave, 0, unroll=False)
    pltpu.sync_copy(ebuf, go_t.at[:, pl.ds(obase, BPW)])


@functools.cache
def _sc_gather():
    return pl.kernel(
        _sc_gather_body,
        out_type=(
            jax.ShapeDtypeStruct((EMB, B), jnp.float32),
            jax.ShapeDtypeStruct((EMB, B), jnp.float32),
            jax.ShapeDtypeStruct((EMB, B), jnp.float32),
            jax.ShapeDtypeStruct((B,), jnp.float32),
        ),
        mesh=plsc.VectorSubcoreMesh(core_axis_name="c", subcore_axis_name="s"),
        compiler_params=pltpu.CompilerParams(use_tc_tiling_on_sc=True,
                                             needs_layout_passes=False),
        scratch_types=(
            pltpu.VMEM((BPW + WAVE,), jnp.int32),
            pltpu.VMEM((2, WAVE, EMB, 128), jnp.float32),
            pltpu.VMEM((EMB, BPW), jnp.float32),
            pltpu.VMEM((EMB, NGENRE), jnp.float32),
            pltpu.VMEM((2, WAVE, 128), jnp.float32),
            pltpu.VMEM((BPW,), jnp.float32),
            pltpu.SemaphoreType.DMA,
            pltpu.SemaphoreType.DMA,
        ),
    )


def _tc_body(ue_t, me_t, ge_t, brow, w1, b1, w2r, b2, out_ref, a_s, b_s):
    i = pl.program_id(0)

    @pl.when(i == 0)
    def _():
        u = ue_t[...]
        m = me_t[...]
        g = ge_t[...]
        dn = (((0,), (0,)), ((), ()))
        h = (lax.dot_general(u, w1[0:EMB, :], dn,
                             preferred_element_type=jnp.float32)
             + lax.dot_general(m, w1[EMB:2 * EMB, :], dn,
                               preferred_element_type=jnp.float32)
             + lax.dot_general(g, w1[2 * EMB:3 * EMB, :], dn,
                               preferred_element_type=jnp.float32)
             + b1[...])
        h = jnp.maximum(h, 0.0)
        a_s[...] = jnp.sum(h * w2r[...], axis=1, keepdims=True) + b2[...]
        wide = jnp.sum(u + m + g, axis=0, keepdims=True)
        b_s[...] = wide + brow[...]

    out_ref[...] = a_s[pl.ds(i * RB, RB), :] + b_s[...]


def _tc_fused(ue_t, me_t, ge_t, brow, w1, b1r, w2r, b2r):
    full = lambda i: (0, 0)
    return pl.pallas_call(
        _tc_body,
        grid=(GRID,),
        in_specs=[
            pl.BlockSpec((EMB, B), full),
            pl.BlockSpec((EMB, B), full),
            pl.BlockSpec((EMB, B), full),
            pl.BlockSpec((1, B), full),
            pl.BlockSpec((3 * EMB, HID), full),
            pl.BlockSpec((1, HID), full),
            pl.BlockSpec((1, HID), full),
            pl.BlockSpec((1, 1), full),
        ],
        out_specs=pl.BlockSpec((RB, B), lambda i: (i, 0)),
        out_shape=jax.ShapeDtypeStruct((B, B), jnp.float32),
        scratch_shapes=[
            pltpu.VMEM((B, 1), jnp.float32),
            pltpu.VMEM((1, B), jnp.float32),
        ],
    )(ue_t, me_t, ge_t, brow, w1, b1r, w2r, b2r)


def kernel(user_ids, movie_ids, genre_ids, user_table, movie_table,
           genre_table, movie_bias_table, W1, b1, W2, b2):
    uid = user_ids.astype(jnp.int32)
    mid = movie_ids.astype(jnp.int32)
    gid = genre_ids.astype(jnp.int32)
    ut_t = user_table.T
    mt_t = movie_table.T
    gt_t = genre_table.T
    bias1d = movie_bias_table.reshape(-1)
    ue_t, me_t, ge_t, bvals = _sc_gather()(uid, mid, gid, ut_t, mt_t, gt_t,
                                           bias1d)
    brow = bvals.reshape(1, B)
    b1r = b1.reshape(1, HID)
    w2r = W2.reshape(1, HID)
    b2r = b2.reshape(1, 1)
    return _tc_fused(ue_t, me_t, ge_t, brow, W1, b1r, w2r, b2r)
